# baseline (speedup 1.0000x reference)
import math
import functools
import jax
import jax.numpy as jnp
from jax.experimental import pallas as pl
from jax.experimental.pallas import tpu as pltpu

N = 2048
D = 256
K = 4
TOPK = 16
CHEB_T = 2
OUT = 256

BLK = 256


def _dot3(a, b):
    return jnp.dot(a.astype(jnp.bfloat16), b.astype(jnp.bfloat16),
                   preferred_element_type=jnp.float32)


def _ln(x):
    mu = x.mean(-1, keepdims=True)
    v = ((x - mu) ** 2).mean(-1, keepdims=True)
    return (x - mu) / jnp.sqrt(v + 1e-5)


_CB = 1024


def _attn_kernel(q_ref, kt_ref, hn_ref, hm_ref, x_ref):
    q = q_ref[...]
    m = jnp.full((BLK, 1), -jnp.inf, jnp.float32)
    l = jnp.zeros((BLK, 1), jnp.float32)
    acc = jnp.zeros((BLK, D), jnp.float32)
    for j in range(N // _CB):
        ktj = kt_ref[j * _CB:(j + 1) * _CB, :]
        vj = hn_ref[j * _CB:(j + 1) * _CB, :]
        sj = _dot3(q, ktj.T) * (1.0 / math.sqrt(D))
        mj = jnp.maximum(m, jnp.max(sj, axis=-1, keepdims=True))
        c = jnp.exp(m - mj)
        p = jnp.exp(sj - mj)
        l = l * c + jnp.sum(p, axis=-1, keepdims=True)
        acc = acc * c + _dot3(p, vj)
        m = mj
    x_ref[...] = acc / l + hm_ref[...]


def _compute_x(H_m, H_n, WQ, bQ, WK, bK):
    Q = _ln(H_m) @ WQ.T + bQ
    Kt = _ln(H_n) @ WK.T + bK

    X = pl.pallas_call(
        _attn_kernel,
        grid=(N // BLK,),
        in_specs=[
            pl.BlockSpec((BLK, D), lambda i: (i, 0)),
            pl.BlockSpec((N, D), lambda i: (0, 0)),
            pl.BlockSpec((N, D), lambda i: (0, 0)),
            pl.BlockSpec((BLK, D), lambda i: (i, 0)),
        ],
        out_specs=pl.BlockSpec((BLK, D), lambda i: (i, 0)),
        out_shape=jax.ShapeDtypeStruct((N, D), jnp.float32),
    )(Q, Kt, H_n, H_m)
    return X


def _rest(X, L_k, alpha, theta, rho_raw, proj_W, proj_b):
    d = D
    n = X.shape[0]
    S = jnp.einsum('nd,kde,me->knm', X, L_k, X) / math.sqrt(d)
    H_stack = jax.nn.softmax(S, axis=2)
    w = jax.nn.softmax(alpha)
    H_fused = (H_stack * w[:, None, None]).sum(0)
    vals, idx = jax.lax.top_k(H_stack, TOPK)
    vals = vals / (vals.sum(-1, keepdims=True) + 1e-9)
    E = K * n
    cols = jnp.broadcast_to(
        (jnp.arange(K) * n)[:, None, None] + jnp.arange(n)[None, :, None],
        idx.shape)
    B = jnp.zeros((n, E), X.dtype).at[idx.reshape(-1), cols.reshape(-1)].set(
        vals.reshape(-1))
    W_e = jnp.ones((E,), X.dtype)
    De = B.sum(0) + 1e-9
    Dv = (B * W_e[None, :]).sum(1) + 1e-9
    dvis = 1.0 / jnp.sqrt(Dv)
    BWDe = B * (W_e / De)[None, :]
    Smat = BWDe @ B.T
    L = jnp.eye(n, dtype=X.dtype) - dvis[:, None] * Smat * dvis[None, :]
    Ld = jax.lax.stop_gradient(L)
    v = jax.random.normal(jax.random.key(1), (n, 1), X.dtype)
    v = v / (jnp.linalg.norm(v) + 1e-9)
    for _ in range(5):
        v = Ld @ v
        v = v / (jnp.linalg.norm(v) + 1e-9)
    lam = jnp.maximum((v.T @ (Ld @ v)).squeeze(), 0.001)
    L_t = (2.0 / lam) * L - jnp.eye(n, dtype=X.dtype)
    T0 = X
    out = T0 * theta[0][None, :]
    T1 = L_t @ X
    out = out + T1 * theta[1][None, :]
    for t in range(2, CHEB_T + 1):
        T2 = 2.0 * (L_t @ T1) - T0
        out = out + T2 * theta[t][None, :]
        T0, T1 = T1, T2
    H_spec = jax.nn.elu(out)
    rho = jax.nn.sigmoid(rho_raw)
    Y = rho * H_spec + (1.0 - rho) * (H_fused @ X)
    return Y @ proj_W.T + proj_b


def kernel(H_m, H_n, WQ, bQ, WK, bK, L_k, alpha, theta, rho_raw, proj_W, proj_b):
    X = _compute_x(H_m, H_n, WQ, bQ, WK, bK)
    return _rest(X, L_k, alpha, theta, rho_raw, proj_W, proj_b)


# speedup vs baseline: 5.9376x; 5.9376x over previous
import math
import jax
import jax.numpy as jnp
from jax import lax
from jax.experimental import pallas as pl
from jax.experimental.pallas import tpu as pltpu

N = 2048
D = 256
K = 4
TOPK = 16
CHEB_T = 2
OUT = 256

BLK = 256
_CB = 1024


def _dot3(a, b):
    return jnp.dot(a.astype(jnp.bfloat16), b.astype(jnp.bfloat16),
                   preferred_element_type=jnp.float32)


def _ln(x):
    mu = x.mean(-1, keepdims=True)
    v = ((x - mu) ** 2).mean(-1, keepdims=True)
    return (x - mu) / jnp.sqrt(v + 1e-5)


def _attn_kernel(q_ref, kt_ref, hn_ref, hm_ref, x_ref):
    q = q_ref[...]
    m = jnp.full((BLK, 1), -jnp.inf, jnp.float32)
    l = jnp.zeros((BLK, 1), jnp.float32)
    acc = jnp.zeros((BLK, D), jnp.float32)
    for j in range(N // _CB):
        ktj = kt_ref[j * _CB:(j + 1) * _CB, :]
        vj = hn_ref[j * _CB:(j + 1) * _CB, :]
        sj = _dot3(q, ktj.T) * (1.0 / math.sqrt(D))
        mj = jnp.maximum(m, jnp.max(sj, axis=-1, keepdims=True))
        c = jnp.exp(m - mj)
        p = jnp.exp(sj - mj)
        l = l * c + jnp.sum(p, axis=-1, keepdims=True)
        acc = acc * c + _dot3(p, vj)
        m = mj
    x_ref[...] = acc / l + hm_ref[...]


def _compute_x(H_m, H_n, WQ, bQ, WK, bK):
    Q = _ln(H_m) @ WQ.T + bQ
    Kt = _ln(H_n) @ WK.T + bK
    X = pl.pallas_call(
        _attn_kernel,
        grid=(N // BLK,),
        in_specs=[
            pl.BlockSpec((BLK, D), lambda i: (i, 0)),
            pl.BlockSpec((N, D), lambda i: (0, 0)),
            pl.BlockSpec((N, D), lambda i: (0, 0)),
            pl.BlockSpec((BLK, D), lambda i: (i, 0)),
        ],
        out_specs=pl.BlockSpec((BLK, D), lambda i: (i, 0)),
        out_shape=jax.ShapeDtypeStruct((N, D), jnp.float32),
    )(Q, Kt, H_n, H_m)
    return X


def _fuse_kernel(xb_ref, xf_ref, lk_ref, w_ref,
                 r_ref, r2_ref, fx_ref, dv_ref, idx_ref, vn_ref, pf_ref):
    i = pl.program_id(0)
    k = pl.program_id(1)
    xb = xb_ref[...]
    xf = xf_ref[...]
    t = _dot3(xb, lk_ref[0])
    s = _dot3(t, xf.T) * (1.0 / math.sqrt(D))
    m = jnp.max(s, axis=-1, keepdims=True)
    p = jnp.exp(s - m)
    z = jnp.sum(p, axis=-1, keepdims=True)
    h = p / z

    colid = lax.broadcasted_iota(jnp.int32, (BLK, N), 1)
    work = h
    mask = jnp.zeros((BLK, N), jnp.bool_)
    idxs = []
    vals = []
    for t_ in range(TOPK):
        v = jnp.max(work, axis=-1, keepdims=True)
        cand = jnp.where(work == v, colid, N)
        am = jnp.min(cand, axis=-1, keepdims=True)
        sel = colid == am
        mask = mask | sel
        work = jnp.where(sel, -1.0, work)
        idxs.append(am)
        vals.append(v)
    idxcat = jnp.concatenate(idxs, axis=1)
    valcat = jnp.concatenate(vals, axis=1)
    s16 = jnp.sum(valcat, axis=-1, keepdims=True)
    denom = s16 + 1e-9
    vn = valcat / denom
    de = s16 / denom + 1e-9
    rv = jnp.where(mask, h, 0.0) / denom
    r_ref[0] = rv.astype(jnp.bfloat16)
    r2_ref[0] = (rv / de).astype(jnp.bfloat16)
    idx_ref[0] = idxcat
    vn_ref[0] = vn

    dvpart = jnp.sum(rv, axis=0, keepdims=True)

    @pl.when(jnp.logical_and(i == 0, k == 0))
    def _():
        dv_ref[...] = jnp.zeros_like(dv_ref)

    dv_ref[...] += dvpart

    wk = jnp.sum(jnp.where(
        lax.broadcasted_iota(jnp.int32, (1, K), 1) == k, w_ref[...], 0.0))

    @pl.when(k == 0)
    def _():
        pf_ref[...] = wk * h

    @pl.when(k > 0)
    def _():
        pf_ref[...] += wk * h

    @pl.when(k == K - 1)
    def _():
        fx_ref[...] = _dot3(pf_ref[...], xf)


def _fuse(X, L_k, w):
    return pl.pallas_call(
        _fuse_kernel,
        grid=(N // BLK, K),
        in_specs=[
            pl.BlockSpec((BLK, D), lambda i, k: (i, 0)),
            pl.BlockSpec((N, D), lambda i, k: (0, 0)),
            pl.BlockSpec((1, D, D), lambda i, k: (k, 0, 0)),
            pl.BlockSpec((1, K), lambda i, k: (0, 0)),
        ],
        out_specs=[
            pl.BlockSpec((1, BLK, N), lambda i, k: (k, i, 0)),
            pl.BlockSpec((1, BLK, N), lambda i, k: (k, i, 0)),
            pl.BlockSpec((BLK, D), lambda i, k: (i, 0)),
            pl.BlockSpec((1, N), lambda i, k: (0, 0)),
            pl.BlockSpec((1, BLK, TOPK), lambda i, k: (k, i, 0)),
            pl.BlockSpec((1, BLK, TOPK), lambda i, k: (k, i, 0)),
        ],
        out_shape=[
            jax.ShapeDtypeStruct((K, N, N), jnp.bfloat16),
            jax.ShapeDtypeStruct((K, N, N), jnp.bfloat16),
            jax.ShapeDtypeStruct((N, D), jnp.float32),
            jax.ShapeDtypeStruct((1, N), jnp.float32),
            jax.ShapeDtypeStruct((K, N, TOPK), jnp.int32),
            jax.ShapeDtypeStruct((K, N, TOPK), jnp.float32),
        ],
        scratch_shapes=[pltpu.VMEM((BLK, N), jnp.float32)],
    )(X, X, L_k, w.reshape(1, K))


def _smat_kernel(r2_ref, r_ref, smat_ref, acc_ref):
    k = pl.program_id(1)
    part = lax.dot_general(r2_ref[0], r_ref[0],
                           (((0,), (0,)), ((), ())),
                           preferred_element_type=jnp.float32)

    @pl.when(k == 0)
    def _():
        acc_ref[...] = part

    @pl.when(k > 0)
    def _():
        acc_ref[...] += part

    @pl.when(k == K - 1)
    def _():
        smat_ref[...] = acc_ref[...]


def _smat(R, R2):
    return pl.pallas_call(
        _smat_kernel,
        grid=(N // BLK, K),
        in_specs=[
            pl.BlockSpec((1, N, BLK), lambda a, k: (k, 0, a)),
            pl.BlockSpec((1, N, N), lambda a, k: (k, 0, 0)),
        ],
        out_specs=pl.BlockSpec((BLK, N), lambda a, k: (a, 0)),
        out_shape=jax.ShapeDtypeStruct((N, N), jnp.float32),
        scratch_shapes=[pltpu.VMEM((BLK, N), jnp.float32)],
    )(R2, R)


def _cheb_kernel(smat_ref, x_ref, fx_ref, dv_ref, v0_ref, th_ref,
                 rho_ref, pw_ref, pb_ref, o_ref):
    dv = dv_ref[...] + 1e-9
    dvis = (1.0 / jnp.sqrt(dv)).T
    smat = smat_ref[...]
    x = x_ref[...]

    def lapply(y):
        z = _dot3(smat, dvis * y)
        return y - dvis * z

    v = v0_ref[...]
    v = v / (jnp.sqrt(jnp.sum(v * v)) + 1e-9)
    for _ in range(5):
        v = lapply(v)
        v = v / (jnp.sqrt(jnp.sum(v * v)) + 1e-9)
    lam = jnp.maximum(jnp.sum(v * lapply(v)), 0.001)
    a = 2.0 / lam

    th0 = th_ref[0, :][None, :]
    th1 = th_ref[1, :][None, :]
    th2 = th_ref[2, :][None, :]
    t1 = a * lapply(x) - x
    out = x * th0 + t1 * th1
    t2 = 2.0 * (a * lapply(t1) - t1) - x
    out = out + t2 * th2
    h_spec = jnp.where(out > 0, out, jnp.exp(out) - 1.0)
    rho = rho_ref[0, 0]
    y = rho * h_spec + (1.0 - rho) * fx_ref[...]
    o_ref[...] = _dot3(y, pw_ref[...].T) + pb_ref[...][None, :]


def _cheb(Smat, X, FX, Dv, v0, theta, rho, proj_W, proj_b):
    return pl.pallas_call(
        _cheb_kernel,
        grid=(1,),
        in_specs=[
            pl.BlockSpec((N, N), lambda i: (0, 0)),
            pl.BlockSpec((N, D), lambda i: (0, 0)),
            pl.BlockSpec((N, D), lambda i: (0, 0)),
            pl.BlockSpec((1, N), lambda i: (0, 0)),
            pl.BlockSpec((N, 1), lambda i: (0, 0)),
            pl.BlockSpec((CHEB_T + 1, D), lambda i: (0, 0)),
            pl.BlockSpec((1, 1), lambda i: (0, 0)),
            pl.BlockSpec((OUT, D), lambda i: (0, 0)),
            pl.BlockSpec((OUT,), lambda i: (0,)),
        ],
        out_specs=pl.BlockSpec((N, OUT), lambda i: (0, 0)),
        out_shape=jax.ShapeDtypeStruct((N, OUT), jnp.float32),
    )(Smat, X, FX, Dv, v0, theta, rho, proj_W, proj_b)


def kernel(H_m, H_n, WQ, bQ, WK, bK, L_k, alpha, theta, rho_raw, proj_W, proj_b):
    X = _compute_x(H_m, H_n, WQ, bQ, WK, bK)
    w = jax.nn.softmax(alpha)
    R, R2, FX, Dv, idx16, vn16 = _fuse(X, L_k, w)
    Smat = _smat(R, R2)
    v0 = jax.random.normal(jax.random.key(1), (N, 1), jnp.float32)
    rho = jax.nn.sigmoid(rho_raw).reshape(1, 1)
    out = _cheb(Smat, X, FX, Dv, v0, theta, rho, proj_W, proj_b)
    return out


# speedup vs baseline: 5.9534x; 1.0027x over previous
import functools
import math
import jax
import jax.numpy as jnp
from jax import lax
from jax.experimental import pallas as pl
from jax.experimental.pallas import tpu as pltpu
from jax.experimental.pallas import tpu_sc as plsc

N = 2048
D = 256
K = 4
TOPK = 16
CHEB_T = 2
OUT = 256

BLK = 256
_CB = 1024


def _dot3(a, b):
    return jnp.dot(a.astype(jnp.bfloat16), b.astype(jnp.bfloat16),
                   preferred_element_type=jnp.float32)


def _ln(x):
    mu = x.mean(-1, keepdims=True)
    v = ((x - mu) ** 2).mean(-1, keepdims=True)
    return (x - mu) / jnp.sqrt(v + 1e-5)


def _attn_kernel(q_ref, kt_ref, hn_ref, hm_ref, x_ref):
    q = q_ref[...]
    m = jnp.full((BLK, 1), -jnp.inf, jnp.float32)
    l = jnp.zeros((BLK, 1), jnp.float32)
    acc = jnp.zeros((BLK, D), jnp.float32)
    for j in range(N // _CB):
        ktj = kt_ref[j * _CB:(j + 1) * _CB, :]
        vj = hn_ref[j * _CB:(j + 1) * _CB, :]
        sj = _dot3(q, ktj.T) * (1.0 / math.sqrt(D))
        mj = jnp.maximum(m, jnp.max(sj, axis=-1, keepdims=True))
        c = jnp.exp(m - mj)
        p = jnp.exp(sj - mj)
        l = l * c + jnp.sum(p, axis=-1, keepdims=True)
        acc = acc * c + _dot3(p, vj)
        m = mj
    x_ref[...] = acc / l + hm_ref[...]


def _compute_x(H_m, H_n, WQ, bQ, WK, bK):
    Q = _ln(H_m) @ WQ.T + bQ
    Kt = _ln(H_n) @ WK.T + bK
    X = pl.pallas_call(
        _attn_kernel,
        grid=(N // BLK,),
        in_specs=[
            pl.BlockSpec((BLK, D), lambda i: (i, 0)),
            pl.BlockSpec((N, D), lambda i: (0, 0)),
            pl.BlockSpec((N, D), lambda i: (0, 0)),
            pl.BlockSpec((BLK, D), lambda i: (i, 0)),
        ],
        out_specs=pl.BlockSpec((BLK, D), lambda i: (i, 0)),
        out_shape=jax.ShapeDtypeStruct((N, D), jnp.float32),
    )(Q, Kt, H_n, H_m)
    return X


def _fuse_kernel(xb_ref, xf_ref, lk_ref, w_ref,
                 r_ref, r2_ref, fx_ref, dv_ref, idx_ref, vn_ref, pf_ref):
    i = pl.program_id(0)
    k = pl.program_id(1)
    xb = xb_ref[...]
    xf = xf_ref[...]
    t = _dot3(xb, lk_ref[0])
    s = _dot3(t, xf.T) * (1.0 / math.sqrt(D))
    m = jnp.max(s, axis=-1, keepdims=True)
    p = jnp.exp(s - m)
    z = jnp.sum(p, axis=-1, keepdims=True)
    h = p / z

    colid = lax.broadcasted_iota(jnp.int32, (BLK, N), 1)
    work = h
    mask = jnp.zeros((BLK, N), jnp.bool_)
    idxs = []
    vals = []
    for t_ in range(TOPK):
        v = jnp.max(work, axis=-1, keepdims=True)
        cand = jnp.where(work == v, colid, N)
        am = jnp.min(cand, axis=-1, keepdims=True)
        sel = colid == am
        mask = mask | sel
        work = jnp.where(sel, -1.0, work)
        idxs.append(am)
        vals.append(v)
    idxcat = jnp.concatenate(idxs, axis=1)
    valcat = jnp.concatenate(vals, axis=1)
    s16 = jnp.sum(valcat, axis=-1, keepdims=True)
    denom = s16 + 1e-9
    vn = valcat / denom
    de = s16 / denom + 1e-9
    rv = jnp.where(mask, h, 0.0) / denom
    r_ref[0] = rv.astype(jnp.bfloat16)
    r2_ref[0] = (rv / de).astype(jnp.bfloat16)
    idx_ref[0] = idxcat
    vn_ref[0] = vn

    dvpart = jnp.sum(rv, axis=0, keepdims=True)

    @pl.when(jnp.logical_and(i == 0, k == 0))
    def _():
        dv_ref[...] = jnp.zeros_like(dv_ref)

    dv_ref[...] += dvpart

    wk = jnp.sum(jnp.where(
        lax.broadcasted_iota(jnp.int32, (1, K), 1) == k, w_ref[...], 0.0))

    @pl.when(k == 0)
    def _():
        pf_ref[...] = wk * h

    @pl.when(k > 0)
    def _():
        pf_ref[...] += wk * h

    @pl.when(k == K - 1)
    def _():
        fx_ref[...] = _dot3(pf_ref[...], xf)


def _fuse(X, L_k, w):
    return pl.pallas_call(
        _fuse_kernel,
        grid=(N // BLK, K),
        in_specs=[
            pl.BlockSpec((BLK, D), lambda i, k: (i, 0)),
            pl.BlockSpec((N, D), lambda i, k: (0, 0)),
            pl.BlockSpec((1, D, D), lambda i, k: (k, 0, 0)),
            pl.BlockSpec((1, K), lambda i, k: (0, 0)),
        ],
        out_specs=[
            pl.BlockSpec((1, BLK, N), lambda i, k: (k, i, 0)),
            pl.BlockSpec((1, BLK, N), lambda i, k: (k, i, 0)),
            pl.BlockSpec((BLK, D), lambda i, k: (i, 0)),
            pl.BlockSpec((1, N), lambda i, k: (0, 0)),
            pl.BlockSpec((1, BLK, TOPK), lambda i, k: (k, i, 0)),
            pl.BlockSpec((1, BLK, TOPK), lambda i, k: (k, i, 0)),
        ],
        out_shape=[
            jax.ShapeDtypeStruct((K, N, N), jnp.bfloat16),
            jax.ShapeDtypeStruct((K, N, N), jnp.bfloat16),
            jax.ShapeDtypeStruct((N, D), jnp.float32),
            jax.ShapeDtypeStruct((1, N), jnp.float32),
            jax.ShapeDtypeStruct((K, N, TOPK), jnp.int32),
            jax.ShapeDtypeStruct((K, N, TOPK), jnp.float32),
        ],
        scratch_shapes=[pltpu.VMEM((BLK, N), jnp.float32)],
    )(X, X, L_k, w.reshape(1, K))


def _smat_kernel(r2_ref, r_ref, smat_ref, acc_ref):
    k = pl.program_id(1)
    part = lax.dot_general(r2_ref[0], r_ref[0],
                           (((0,), (0,)), ((), ())),
                           preferred_element_type=jnp.float32)

    @pl.when(k == 0)
    def _():
        acc_ref[...] = part

    @pl.when(k > 0)
    def _():
        acc_ref[...] += part

    @pl.when(k == K - 1)
    def _():
        smat_ref[...] = acc_ref[...]


def _smat(R, R2):
    return pl.pallas_call(
        _smat_kernel,
        grid=(N // BLK, K),
        in_specs=[
            pl.BlockSpec((1, N, BLK), lambda a, k: (k, 0, a)),
            pl.BlockSpec((1, N, N), lambda a, k: (k, 0, 0)),
        ],
        out_specs=pl.BlockSpec((BLK, N), lambda a, k: (a, 0)),
        out_shape=jax.ShapeDtypeStruct((N, N), jnp.float32),
        scratch_shapes=[pltpu.VMEM((BLK, N), jnp.float32)],
    )(R2, R)


_E = K * N
_EPW = _E // 16
_NCH = N // 16


def _ssum(x):
    lane = lax.broadcasted_iota(jnp.int32, (16,), 0)
    for sh in (8, 4, 2, 1):
        x = x + x.at[(lane + sh) & 15].get(mode='promise_in_bounds')
    return x


def _power_sc_kernel(idx_hbm, vn_hbm, dvis_hbm, v0_hbm, zid_hbm, out_hbm,
                     outu_hbm, idx_v, vn_v, y_v, yd_v, outp_v, outp2_v, dvis_v,
                     comb_v, zid_v, out16_v, acc_sh):
    sid = lax.axis_index("s")
    cid = lax.axis_index("c")
    wid = sid + cid * 16
    base = sid * _EPW
    zero16 = jnp.zeros((16,), jnp.float32)
    zidx16 = jnp.zeros((16,), jnp.int32)

    pltpu.sync_copy(idx_hbm.at[pl.ds(base * TOPK, _EPW * TOPK)], idx_v)
    pltpu.sync_copy(vn_hbm.at[pl.ds(base * TOPK, _EPW * TOPK)], vn_v)
    pltpu.sync_copy(dvis_hbm, dvis_v)
    pltpu.sync_copy(v0_hbm, y_v)
    pltpu.sync_copy(zid_hbm, zid_v)

    def do_apply():
        def prep(c, _):
            sl = pl.ds(c * 16, 16)
            yd_v[sl] = dvis_v[0, sl] * y_v[0, sl]
            outp_v[sl] = zero16
            return 0
        lax.fori_loop(0, _NCH, prep, 0)

        def ebody(e, _):
            sl = pl.ds(e * TOPK, TOPK)
            iv = idx_v[sl]
            vv = vn_v[sl]
            g = plsc.load_gather(yd_v, [iv])
            z = _ssum(vv * g)
            q = vv / (_ssum(vv) + 1e-9)
            plsc.addupdate_scatter(outp_v, [iv], q * z)
            return 0
        lax.fori_loop(0, _EPW, ebody, 0)

        def stage(c, _):
            sl = pl.ds(c * 16, 16)
            outp2_v[0, sl] = outp_v[sl]
            return 0
        lax.fori_loop(0, _NCH, stage, 0)

        @pl.when(sid == 0)
        def _():
            def zsh(c, _):
                comb_v[0, pl.ds(c * 16, 16)] = zero16
                return 0
            lax.fori_loop(0, _NCH, zsh, 0)
            pltpu.sync_copy(comb_v, acc_sh)
        plsc.subcore_barrier()
        pltpu.sync_copy(outp2_v, acc_sh.at[zid_v], add=True)
        plsc.subcore_barrier()
        pltpu.sync_copy(acc_sh, comb_v)

    for _r in range(5):
        do_apply()

        def upd(c, _):
            sl = pl.ds(c * 16, 16)
            y_v[0, sl] = y_v[0, sl] - dvis_v[0, sl] * comb_v[0, sl]
            return 0
        lax.fori_loop(0, _NCH, upd, 0)
        plsc.subcore_barrier()

    do_apply()

    def rq(c, carry):
        num, den = carry
        sl = pl.ds(c * 16, 16)
        u = y_v[0, sl]
        w = u - dvis_v[0, sl] * comb_v[0, sl]
        return num + u * w, den + u * u
    zv = jnp.zeros((16,), jnp.float32)
    num, den = lax.fori_loop(0, _NCH, rq, (zv, zv))
    num = _ssum(num)
    den = _ssum(den)

    @pl.when(wid == 0)
    def _():
        lane = lax.broadcasted_iota(jnp.int32, (16,), 0)
        out16_v[...] = jnp.where(lane == 0, num,
                                 jnp.where(lane == 1, den, 0.0))
        pltpu.sync_copy(out16_v, out_hbm)
        pltpu.sync_copy(y_v, outu_hbm)


def _power_sc(idx16, vn16, dvis, v0n):
    mesh = plsc.VectorSubcoreMesh(core_axis_name="c", subcore_axis_name="s")
    kfn = functools.partial(
        pl.kernel, mesh=mesh,
        compiler_params=pltpu.CompilerParams(needs_layout_passes=False),
        out_type=[jax.ShapeDtypeStruct((16,), jnp.float32),
                  jax.ShapeDtypeStruct((1, N), jnp.float32)],
        scratch_types=[
            pltpu.VMEM((_EPW * TOPK,), jnp.int32),
            pltpu.VMEM((_EPW * TOPK,), jnp.float32),
            pltpu.VMEM((1, N), jnp.float32),
            pltpu.VMEM((N,), jnp.float32),
            pltpu.VMEM((N,), jnp.float32),
            pltpu.VMEM((1, N), jnp.float32),
            pltpu.VMEM((1, N), jnp.float32),
            pltpu.VMEM((1, N), jnp.float32),
            pltpu.VMEM((1,), jnp.int32),
            pltpu.VMEM((16,), jnp.float32),
            pltpu.VMEM_SHARED((1, N), jnp.float32),
        ],
    )(_power_sc_kernel)
    zid = jnp.zeros((1,), jnp.int32)
    out16, u = kfn(idx16.reshape(_E * TOPK), vn16.reshape(_E * TOPK),
                   dvis.reshape(1, N), v0n.reshape(1, N), zid)
    num = out16[0]
    den = out16[1]
    return jnp.maximum(num / den, 0.001), u


def _cheb_kernel(smat_ref, x_ref, fx_ref, dv_ref, lam_ref, th_ref,
                 rho_ref, pw_ref, pb_ref, o_ref):
    dv = dv_ref[...] + 1e-9
    dvis = (1.0 / jnp.sqrt(dv)).T
    smat = smat_ref[...]
    x = x_ref[...]

    def lapply(y):
        z = _dot3(smat, dvis * y)
        return y - dvis * z

    lam = lam_ref[0, 0]
    a = 2.0 / lam

    th0 = th_ref[0, :][None, :]
    th1 = th_ref[1, :][None, :]
    th2 = th_ref[2, :][None, :]
    t1 = a * lapply(x) - x
    out = x * th0 + t1 * th1
    t2 = 2.0 * (a * lapply(t1) - t1) - x
    out = out + t2 * th2
    h_spec = jnp.where(out > 0, out, jnp.exp(out) - 1.0)
    rho = rho_ref[0, 0]
    y = rho * h_spec + (1.0 - rho) * fx_ref[...]
    o_ref[...] = _dot3(y, pw_ref[...].T) + pb_ref[...][None, :]


def _cheb(Smat, X, FX, Dv, lam, theta, rho, proj_W, proj_b):
    return pl.pallas_call(
        _cheb_kernel,
        grid=(1,),
        in_specs=[
            pl.BlockSpec((N, N), lambda i: (0, 0)),
            pl.BlockSpec((N, D), lambda i: (0, 0)),
            pl.BlockSpec((N, D), lambda i: (0, 0)),
            pl.BlockSpec((1, N), lambda i: (0, 0)),
            pl.BlockSpec((1, 1), lambda i: (0, 0)),
            pl.BlockSpec((CHEB_T + 1, D), lambda i: (0, 0)),
            pl.BlockSpec((1, 1), lambda i: (0, 0)),
            pl.BlockSpec((OUT, D), lambda i: (0, 0)),
            pl.BlockSpec((OUT,), lambda i: (0,)),
        ],
        out_specs=pl.BlockSpec((N, OUT), lambda i: (0, 0)),
        out_shape=jax.ShapeDtypeStruct((N, OUT), jnp.float32),
    )(Smat, X, FX, Dv, lam, theta, rho, proj_W, proj_b)


def kernel(H_m, H_n, WQ, bQ, WK, bK, L_k, alpha, theta, rho_raw, proj_W, proj_b):
    X = _compute_x(H_m, H_n, WQ, bQ, WK, bK)
    w = jax.nn.softmax(alpha)
    R, R2, FX, Dv, idx16, vn16 = _fuse(X, L_k, w)
    Smat = _smat(R, R2)
    v0 = jax.random.normal(jax.random.key(1), (N, 1), jnp.float32)
    v0n = v0 / (jnp.linalg.norm(v0) + 1e-9)
    dvis = 1.0 / jnp.sqrt(Dv.reshape(N) + 1e-9)
    lam = _power_sc(idx16, vn16, dvis, v0n)[0].reshape(1, 1)
    rho = jax.nn.sigmoid(rho_raw).reshape(1, 1)
    out = _cheb(Smat, X, FX, Dv, lam, theta, rho, proj_W, proj_b)
    return out
